# baseline (speedup 1.0000x reference)
import jax
import jax.numpy as jnp
import numpy as np
from jax import lax
from jax.experimental import pallas as pl
from jax.experimental.pallas import tpu as pltpu
from jax.experimental.pallas import tpu_sc as plsc

_RCUT = 6.0
_RCUT_SMTH = 0.5
_NLOC = 10000
_NALL = 12000
_NNEI = 64
_NSEL = 32
_NG = 100
_AXIS = 16
_REP = 1
_E = _NLOC * _NNEI

_NC, _NS, _L = 2, 16, 16
_NW = _NC * _NS
_PER_W = _E // _NW
_CH = 2000

_BA = 80
_NB = _NLOC // _BA
_R = _BA * _NSEL


def _sc_rsqrt(x):
    i = plsc.bitcast(x, jnp.int32)
    i = jnp.int32(0x5F3759DF) - (i >> 1)
    y = plsc.bitcast(i, jnp.float32)
    for _ in range(3):
        y = y * (1.5 - 0.5 * x * y * y)
    return y


def _sc_env_body(nlist_hbm, xt_hbm, yt_hbm, zt_hbm, at_hbm, mean_hbm, istd_hbm,
                 r0_hbm, r1_hbm, r2_hbm, r3_hbm, sw_hbm,
                 xt_v, yt_v, zt_v, at_v, mean_v, istd_v,
                 nl_v, b0_v, b1_v, b2_v, b3_v, osw_v):
    wid = lax.axis_index("s") * _NC + lax.axis_index("c")
    pltpu.sync_copy(xt_hbm, xt_v)
    pltpu.sync_copy(yt_hbm, yt_v)
    pltpu.sync_copy(zt_hbm, zt_v)
    pltpu.sync_copy(at_hbm, at_v)
    pltpu.sync_copy(mean_hbm, mean_v)
    pltpu.sync_copy(istd_hbm, istd_v)

    inv_span = 1.0 / (_RCUT - _RCUT_SMTH)

    for ch in range(_PER_W // _CH):
        base = wid * _PER_W + ch * _CH
        pltpu.sync_copy(nlist_hbm.at[pl.ds(base, _CH)], nl_v)

        def body(i, carry):
            off = i * _L
            idx = nl_v[pl.ds(off, _L)]
            e = base + off + lax.iota(jnp.int32, _L)
            aid = e >> 6
            slot = e & 63
            xr = plsc.load_gather(xt_v, [idx])
            yr = plsc.load_gather(yt_v, [idx])
            zr = plsc.load_gather(zt_v, [idx])
            xl = plsc.load_gather(xt_v, [aid])
            yl = plsc.load_gather(yt_v, [aid])
            zl = plsc.load_gather(zt_v, [aid])
            dx = xr - xl
            dy = yr - yl
            dz = zr - zl
            r2 = dx * dx + dy * dy + dz * dz
            rinv = _sc_rsqrt(r2)
            r = r2 * rinv
            r2inv = rinv * rinv
            uu = (r - _RCUT_SMTH) * inv_span
            vv = uu * uu * uu * (-6.0 * uu * uu + 15.0 * uu - 10.0) + 1.0
            sw = jnp.where(r <= _RCUT_SMTH, 1.0, jnp.where(r >= _RCUT, 0.0, vv))
            e0 = rinv * sw
            e1 = dx * r2inv * sw
            e2 = dy * r2inv * sw
            e3 = dz * r2inv * sw
            t = plsc.load_gather(at_v, [aid])
            mb = t * 256 + slot * 4
            d0 = (e0 - plsc.load_gather(mean_v, [mb])) * plsc.load_gather(istd_v, [mb])
            d1 = (e1 - plsc.load_gather(mean_v, [mb + 1])) * plsc.load_gather(istd_v, [mb + 1])
            d2 = (e2 - plsc.load_gather(mean_v, [mb + 2])) * plsc.load_gather(istd_v, [mb + 2])
            d3 = (e3 - plsc.load_gather(mean_v, [mb + 3])) * plsc.load_gather(istd_v, [mb + 3])
            osw_v[pl.ds(off, _L)] = sw
            b0_v[pl.ds(off, _L)] = d0
            b1_v[pl.ds(off, _L)] = d1
            b2_v[pl.ds(off, _L)] = d2
            b3_v[pl.ds(off, _L)] = d3
            return carry

        lax.fori_loop(0, _CH // _L, body, 0)
        rsl = pl.ds(base * _REP, _CH * _REP)
        pltpu.sync_copy(b0_v, r0_hbm.at[rsl])
        pltpu.sync_copy(b1_v, r1_hbm.at[rsl])
        pltpu.sync_copy(b2_v, r2_hbm.at[rsl])
        pltpu.sync_copy(b3_v, r3_hbm.at[rsl])
        pltpu.sync_copy(osw_v, sw_hbm.at[pl.ds(base, _CH)])


def _sc_env(nlist_flat, xt, yt, zt, at, mean_flat, istd_flat):
    f32 = jnp.float32
    out_type = [jax.ShapeDtypeStruct((_E * _REP,), f32) for _ in range(4)]
    out_type.append(jax.ShapeDtypeStruct((_E,), f32))
    scratch = [
        pltpu.VMEM((_NALL,), f32), pltpu.VMEM((_NALL,), f32),
        pltpu.VMEM((_NALL,), f32), pltpu.VMEM((_NALL,), jnp.int32),
        pltpu.VMEM((512,), f32), pltpu.VMEM((512,), f32),
        pltpu.VMEM((_CH,), jnp.int32),
        pltpu.VMEM((_CH * _REP,), f32), pltpu.VMEM((_CH * _REP,), f32),
        pltpu.VMEM((_CH * _REP,), f32), pltpu.VMEM((_CH * _REP,), f32),
        pltpu.VMEM((_CH,), f32),
    ]
    mesh = plsc.VectorSubcoreMesh(core_axis_name="c", subcore_axis_name="s")
    fn = pl.kernel(_sc_env_body, out_type=out_type, mesh=mesh,
                   scratch_types=scratch,
                   compiler_params=pltpu.CompilerParams(
                       needs_layout_passes=False))
    return fn(nlist_flat, xt, yt, zt, at, mean_flat, istd_flat)


def _make_finalize_consts():
    cab = np.zeros((_NG, 2 * _NG * _AXIS), np.float32)
    k = np.arange(_NG)
    j = np.arange(_AXIS)
    cab[k[:, None], k[:, None] * _AXIS + j[None, :]] = 1.0 / _NNEI
    for jj in range(_AXIS):
        cab[jj, _NG * _AXIS + k * _AXIS + jj] = 1.0 / _NNEI
    tr3 = np.zeros((3, _NG, 3 * _NG), np.float32)
    for c in range(3):
        tr3[c, k, k * 3 + c] = 1.0 / _NNEI
    return jnp.asarray(cab), jnp.asarray(tr3)


def _tc_body(rep0, rep1, rep2, rep3, w1, b1, w2, b2, w3, b3, cab, tr3,
             res, rot):
    reps = [rep0, rep1, rep2, rep3]
    s_acc = [jnp.zeros((_BA, _NG), jnp.float32) for _ in range(4)]
    for seg in range(2):
        sl = pl.ds(seg * _NSEL, _NSEL)
        ss1 = rep0[:, sl, 0:1]
        y1 = jnp.tanh(ss1 * w1[seg].reshape(1, 1, 25)
                      + b1[seg].reshape(1, 1, 25))
        y1 = jnp.reshape(y1, (_R, 25))
        y2 = jnp.tanh(jax.lax.dot(y1, w2[seg]) + b2[seg].reshape(1, 50))
        y2 = y2 + jnp.concatenate([y1, y1], axis=1)
        gg = jnp.tanh(jax.lax.dot(y2, w3[seg]) + b3[seg].reshape(1, 100))
        gg = gg + jnp.concatenate([y2, y2], axis=1)
        gg3 = jnp.reshape(gg, (_BA, _NSEL, _NG))
        for c in range(4):
            prod = gg3 * reps[c][:, sl, 0:1]
            s_acc[c] = s_acc[c] + jnp.sum(prod, axis=1)
    acc = None
    for c in range(4):
        ab = jax.lax.dot(s_acc[c], cab[...])
        term = ab[:, : _NG * _AXIS] * ab[:, _NG * _AXIS:]
        acc = term if acc is None else acc + term
    res[...] = acc
    rot[...] = (jax.lax.dot(s_acc[1], tr3[0])
                + jax.lax.dot(s_acc[2], tr3[1])
                + jax.lax.dot(s_acc[3], tr3[2]))


def _tc_mlp(rep0, rep1, rep2, rep3, w1, b1, w2, b2, w3, b3):
    f32 = jnp.float32
    cab, tr3 = _make_finalize_consts()
    rblk = pl.BlockSpec((_BA, _NNEI, _REP), lambda i: (i, 0, 0))
    whole = lambda *s: pl.BlockSpec(s, lambda i, _s=s: tuple(0 for _ in _s))
    return pl.pallas_call(
        _tc_body,
        grid=(_NB,),
        in_specs=[
            rblk, rblk, rblk, rblk,
            whole(2, 25), whole(2, 25),
            whole(2, 25, 50), whole(2, 50),
            whole(2, 50, 100), whole(2, 100),
            whole(_NG, 2 * _NG * _AXIS), whole(3, _NG, 3 * _NG),
        ],
        out_specs=[
            pl.BlockSpec((_BA, _NG * _AXIS), lambda i: (i, 0)),
            pl.BlockSpec((_BA, 3 * _NG), lambda i: (i, 0)),
        ],
        out_shape=[
            jax.ShapeDtypeStruct((_NLOC, _NG * _AXIS), f32),
            jax.ShapeDtypeStruct((_NLOC, 3 * _NG), f32),
        ],
    )(rep0, rep1, rep2, rep3, w1, b1, w2, b2, w3, b3, cab, tr3)


def kernel(nlist, extended_coord, extended_atype, mean, stddev, W1, b1, W2, b2, W3, b3):
    nf, nloc, nnei = nlist.shape
    coord = extended_coord.reshape(_NALL, 3)
    xt = coord[:, 0]
    yt = coord[:, 1]
    zt = coord[:, 2]
    at = extended_atype.reshape(_NALL)
    mean_flat = mean.reshape(512)
    istd_flat = (1.0 / stddev).reshape(512)
    nlist_flat = nlist.reshape(_E)

    r0, r1, r2, r3, swf = _sc_env(
        nlist_flat, xt, yt, zt, at, mean_flat, istd_flat)

    res2, rot2 = _tc_mlp(
        r0.reshape(_NLOC, _NNEI, _REP), r1.reshape(_NLOC, _NNEI, _REP),
        r2.reshape(_NLOC, _NNEI, _REP), r3.reshape(_NLOC, _NNEI, _REP),
        W1.reshape(2, 25), b1, W2, b2, W3, b3)

    result = res2.reshape(nf, nloc, _NG * _AXIS)
    rot_mat = rot2.reshape(nf, nloc, _NG, 3)
    sw = swf.reshape(nf, nloc, nnei, 1)
    return result, rot_mat, sw


# speedup vs baseline: 1.4250x; 1.4250x over previous
import jax
import jax.numpy as jnp
import numpy as np
from jax import lax
from jax.experimental import pallas as pl
from jax.experimental.pallas import tpu as pltpu
from jax.experimental.pallas import tpu_sc as plsc

_RCUT = 6.0
_RCUT_SMTH = 0.5
_NLOC = 10000
_NALL = 12000
_NNEI = 64
_NSEL = 32
_NG = 100
_AXIS = 16
_REP = 8
_E = _NLOC * _NNEI

_NC, _NS, _L = 2, 16, 16
_NW = _NC * _NS
_PER_W = _E // _NW
_CH = 2000

_BA = 80
_NB = _NLOC // _BA
_R = _BA * _NSEL


def _sc_rsqrt(x):
    i = plsc.bitcast(x, jnp.int32)
    i = jnp.int32(0x5F3759DF) - (i >> 1)
    y = plsc.bitcast(i, jnp.float32)
    for _ in range(3):
        y = y * (1.5 - 0.5 * x * y * y)
    return y


def _sc_env_body(nlist_hbm, xt_hbm, yt_hbm, zt_hbm, at_hbm, mean_hbm, istd_hbm,
                 rr_hbm, sw_hbm,
                 xt_v, yt_v, zt_v, at_v, mean_v, istd_v,
                 nl_v, br_v, osw_v):
    wid = lax.axis_index("s") * _NC + lax.axis_index("c")
    pltpu.sync_copy(xt_hbm, xt_v)
    pltpu.sync_copy(yt_hbm, yt_v)
    pltpu.sync_copy(zt_hbm, zt_v)
    pltpu.sync_copy(at_hbm, at_v)
    pltpu.sync_copy(mean_hbm, mean_v)
    pltpu.sync_copy(istd_hbm, istd_v)

    inv_span = 1.0 / (_RCUT - _RCUT_SMTH)
    iota_rep = lax.iota(jnp.int32, _L) * (4 * _REP)

    for ch in range(_PER_W // _CH):
        base = wid * _PER_W + ch * _CH
        pltpu.sync_copy(nlist_hbm.at[pl.ds(base, _CH)], nl_v)

        def body(i, carry):
            off = i * _L
            idx = nl_v[pl.ds(off, _L)]
            e = base + off + lax.iota(jnp.int32, _L)
            aid = e >> 6
            slot = e & 63
            xr = plsc.load_gather(xt_v, [idx])
            yr = plsc.load_gather(yt_v, [idx])
            zr = plsc.load_gather(zt_v, [idx])
            xl = plsc.load_gather(xt_v, [aid])
            yl = plsc.load_gather(yt_v, [aid])
            zl = plsc.load_gather(zt_v, [aid])
            dx = xr - xl
            dy = yr - yl
            dz = zr - zl
            r2 = dx * dx + dy * dy + dz * dz
            rinv = _sc_rsqrt(r2)
            r = r2 * rinv
            r2inv = rinv * rinv
            uu = (r - _RCUT_SMTH) * inv_span
            vv = uu * uu * uu * (-6.0 * uu * uu + 15.0 * uu - 10.0) + 1.0
            sw = jnp.where(r <= _RCUT_SMTH, 1.0, jnp.where(r >= _RCUT, 0.0, vv))
            e0 = rinv * sw
            e1 = dx * r2inv * sw
            e2 = dy * r2inv * sw
            e3 = dz * r2inv * sw
            t = plsc.load_gather(at_v, [aid])
            mb = t * 256 + slot * 4
            d0 = (e0 - plsc.load_gather(mean_v, [mb])) * plsc.load_gather(istd_v, [mb])
            d1 = (e1 - plsc.load_gather(mean_v, [mb + 1])) * plsc.load_gather(istd_v, [mb + 1])
            d2 = (e2 - plsc.load_gather(mean_v, [mb + 2])) * plsc.load_gather(istd_v, [mb + 2])
            d3 = (e3 - plsc.load_gather(mean_v, [mb + 3])) * plsc.load_gather(istd_v, [mb + 3])
            osw_v[pl.ds(off, _L)] = sw
            rbase = off * (4 * _REP) + iota_rep
            for l in range(_REP):
                plsc.store_scatter(br_v, [rbase + l], d0)
                plsc.store_scatter(br_v, [rbase + _REP + l], d1)
                plsc.store_scatter(br_v, [rbase + 2 * _REP + l], d2)
                plsc.store_scatter(br_v, [rbase + 3 * _REP + l], d3)
            return carry

        lax.fori_loop(0, _CH // _L, body, 0)
        rsl = pl.ds(base * 4 * _REP, _CH * 4 * _REP)
        pltpu.sync_copy(br_v, rr_hbm.at[rsl])
        pltpu.sync_copy(osw_v, sw_hbm.at[pl.ds(base, _CH)])


def _sc_env(nlist_flat, xt, yt, zt, at, mean_flat, istd_flat):
    f32 = jnp.float32
    out_type = [jax.ShapeDtypeStruct((_E * 4 * _REP,), f32),
                jax.ShapeDtypeStruct((_E,), f32)]
    scratch = [
        pltpu.VMEM((_NALL,), f32), pltpu.VMEM((_NALL,), f32),
        pltpu.VMEM((_NALL,), f32), pltpu.VMEM((_NALL,), jnp.int32),
        pltpu.VMEM((512,), f32), pltpu.VMEM((512,), f32),
        pltpu.VMEM((_CH,), jnp.int32),
        pltpu.VMEM((_CH * 4 * _REP,), f32),
        pltpu.VMEM((_CH,), f32),
    ]
    mesh = plsc.VectorSubcoreMesh(core_axis_name="c", subcore_axis_name="s")
    fn = pl.kernel(_sc_env_body, out_type=out_type, mesh=mesh,
                   scratch_types=scratch,
                   compiler_params=pltpu.CompilerParams(
                       needs_layout_passes=False))
    return fn(nlist_flat, xt, yt, zt, at, mean_flat, istd_flat)


def _make_finalize_consts():
    cab = np.zeros((_NG, 2 * _NG * _AXIS), np.float32)
    k = np.arange(_NG)
    j = np.arange(_AXIS)
    cab[k[:, None], k[:, None] * _AXIS + j[None, :]] = 1.0 / _NNEI
    for jj in range(_AXIS):
        cab[jj, _NG * _AXIS + k * _AXIS + jj] = 1.0 / _NNEI
    tr3 = np.zeros((3, _NG, 3 * _NG), np.float32)
    for c in range(3):
        tr3[c, k, k * 3 + c] = 1.0 / _NNEI
    return jnp.asarray(cab), jnp.asarray(tr3)


def _tc_body(repall, w1, b1, w2, b2, w3, b3, cab, tr3, res, rot):
    s_acc = [jnp.zeros((_BA, _NG), jnp.float32) for _ in range(4)]
    for seg in range(2):
        sl = pl.ds(seg * _NSEL, _NSEL)
        ss1 = repall[:, sl, 0:1]
        y1 = jnp.tanh(ss1 * w1[seg].reshape(1, 1, 25)
                      + b1[seg].reshape(1, 1, 25))
        y1 = jnp.reshape(y1, (_R, 25))
        y2 = jnp.tanh(jax.lax.dot(y1, w2[seg]) + b2[seg].reshape(1, 50))
        y2 = y2 + jnp.concatenate([y1, y1], axis=1)
        gg = jnp.tanh(jax.lax.dot(y2, w3[seg]) + b3[seg].reshape(1, 100))
        gg = gg + jnp.concatenate([y2, y2], axis=1)
        gg3 = jnp.reshape(gg, (_BA, _NSEL, _NG))
        for c in range(4):
            prod = gg3 * repall[:, sl, c * _REP:c * _REP + 1]
            s_acc[c] = s_acc[c] + jnp.sum(prod, axis=1)
    acc = None
    for c in range(4):
        ab = jax.lax.dot(s_acc[c], cab[...])
        term = ab[:, : _NG * _AXIS] * ab[:, _NG * _AXIS:]
        acc = term if acc is None else acc + term
    res[...] = acc
    rot[...] = (jax.lax.dot(s_acc[1], tr3[0])
                + jax.lax.dot(s_acc[2], tr3[1])
                + jax.lax.dot(s_acc[3], tr3[2]))


def _tc_mlp(repall, w1, b1, w2, b2, w3, b3):
    f32 = jnp.float32
    cab, tr3 = _make_finalize_consts()
    rblk = pl.BlockSpec((_BA, _NNEI, 4 * _REP), lambda i: (i, 0, 0))
    whole = lambda *s: pl.BlockSpec(s, lambda i, _s=s: tuple(0 for _ in _s))
    return pl.pallas_call(
        _tc_body,
        grid=(_NB,),
        in_specs=[
            rblk,
            whole(2, 25), whole(2, 25),
            whole(2, 25, 50), whole(2, 50),
            whole(2, 50, 100), whole(2, 100),
            whole(_NG, 2 * _NG * _AXIS), whole(3, _NG, 3 * _NG),
        ],
        out_specs=[
            pl.BlockSpec((_BA, _NG * _AXIS), lambda i: (i, 0)),
            pl.BlockSpec((_BA, 3 * _NG), lambda i: (i, 0)),
        ],
        out_shape=[
            jax.ShapeDtypeStruct((_NLOC, _NG * _AXIS), f32),
            jax.ShapeDtypeStruct((_NLOC, 3 * _NG), f32),
        ],
    )(repall, w1, b1, w2, b2, w3, b3, cab, tr3)


def kernel(nlist, extended_coord, extended_atype, mean, stddev, W1, b1, W2, b2, W3, b3):
    nf, nloc, nnei = nlist.shape
    coord = extended_coord.reshape(_NALL, 3)
    xt = coord[:, 0]
    yt = coord[:, 1]
    zt = coord[:, 2]
    at = extended_atype.reshape(_NALL)
    mean_flat = mean.reshape(512)
    istd_flat = (1.0 / stddev).reshape(512)
    nlist_flat = nlist.reshape(_E)

    rall, swf = _sc_env(
        nlist_flat, xt, yt, zt, at, mean_flat, istd_flat)

    res2, rot2 = _tc_mlp(
        rall.reshape(_NLOC, _NNEI, 4 * _REP),
        W1.reshape(2, 25), b1, W2, b2, W3, b3)

    result = res2.reshape(nf, nloc, _NG * _AXIS)
    rot_mat = rot2.reshape(nf, nloc, _NG, 3)
    sw = swf.reshape(nf, nloc, nnei, 1)
    return result, rot_mat, sw
